# Initial kernel scaffold; baseline (speedup 1.0000x reference)
import functools

import jax
import jax.numpy as jnp
from jax import lax
from jax.experimental import pallas as pl
from jax.experimental.pallas import tpu as pltpu
from jax.experimental.pallas import tpu_sc as plsc

NC = 2
NS = 16
NW = NC * NS
LANES = 16
CHUNK = 100


def _build(n_flat, d_model, seq_len):
    n_chunks = n_flat // CHUNK
    cpw = n_chunks // NW
    mesh = plsc.VectorSubcoreMesh(core_axis_name="c", subcore_axis_name="s")

    @functools.partial(
        pl.kernel,
        out_type=jax.ShapeDtypeStruct((n_flat, d_model), jnp.float32),
        mesh=mesh,
        scratch_types=[
            pltpu.VMEM((CHUNK,), jnp.int32),
            pltpu.VMEM((CHUNK, d_model), jnp.float32),
            pltpu.VMEM((seq_len, d_model), jnp.float32),
            pltpu.SemaphoreType.DMA,
        ],
    )
    def emb(tok_hbm, table_hbm, pos_hbm, out_hbm, idx_v, rows_v, pos_v, sem):
        wid = lax.axis_index("s") * NC + lax.axis_index("c")
        base = wid * cpw
        pltpu.sync_copy(pos_hbm, pos_v)

        def do_chunk(g, pos_off):
            pltpu.sync_copy(tok_hbm.at[g], idx_v)
            pltpu.async_copy(table_hbm.at[idx_v], rows_v, sem).wait()

            def add_rows(j, _):
                for r in range(4):
                    row = j * 4 + r
                    for k in range(d_model // LANES):
                        plsc.addupdate(
                            rows_v.at[row, pl.ds(k * LANES, LANES)],
                            pos_v[pos_off + row, pl.ds(k * LANES, LANES)],
                        )
                return 0

            lax.fori_loop(0, CHUNK // 4, add_rows, 0)
            pltpu.sync_copy(rows_v, out_hbm.at[pl.ds(g * CHUNK, CHUNK)])

        def body(c2, _):
            g = base + c2 * 2
            do_chunk(g, 0)
            do_chunk(g + 1, CHUNK)
            return 0

        lax.fori_loop(0, cpw // 2, body, 0)

    return emb


def kernel(tokens, token_table, position_table):
    b, s = tokens.shape
    _, d_model = token_table.shape
    n_flat = b * s
    emb = _build(n_flat, d_model, s)
    flat_tok = tokens.reshape(n_flat // CHUNK, CHUNK).astype(jnp.int32)
    out = emb(flat_tok, token_table, position_table)
    return out.reshape(b, s, d_model)


# speedup vs baseline: 5.2106x; 5.2106x over previous
import functools

import jax
import jax.numpy as jnp
from jax import lax
from jax.experimental import pallas as pl
from jax.experimental.pallas import tpu as pltpu
from jax.experimental.pallas import tpu_sc as plsc

NC = 2
NS = 16
NW = NC * NS
LANES = 16
CHUNK = 100


def _build(n_flat, d_model, seq_len):
    n_chunks = n_flat // CHUNK
    cpw = n_chunks // NW
    mesh = plsc.VectorSubcoreMesh(core_axis_name="c", subcore_axis_name="s")

    @functools.partial(
        pl.kernel,
        out_type=jax.ShapeDtypeStruct((n_flat, d_model), jnp.float32),
        mesh=mesh,
        compiler_params=pltpu.CompilerParams(use_tc_tiling_on_sc=False),
        scratch_types=[
            pltpu.VMEM((CHUNK,), jnp.int32),
            pltpu.VMEM((CHUNK, d_model), jnp.float32),
            pltpu.VMEM((seq_len, d_model), jnp.float32),
            pltpu.SemaphoreType.DMA,
        ],
    )
    def emb(tok_hbm, table_hbm, pos_hbm, out_hbm, idx_v, rows_v, pos_v, sem):
        wid = lax.axis_index("s") * NC + lax.axis_index("c")
        base = wid * cpw
        pltpu.sync_copy(pos_hbm, pos_v)

        def do_chunk(g, pos_off):
            pltpu.sync_copy(tok_hbm.at[g], idx_v)
            pltpu.async_copy(table_hbm.at[idx_v], rows_v, sem).wait()

            def add_rows(j, _):
                for r in range(4):
                    row = j * 4 + r
                    for k in range(d_model // LANES):
                        plsc.addupdate(
                            rows_v.at[row, pl.ds(k * LANES, LANES)],
                            pos_v[pos_off + row, pl.ds(k * LANES, LANES)],
                        )
                return 0

            lax.fori_loop(0, CHUNK // 4, add_rows, 0)
            pltpu.sync_copy(rows_v, out_hbm.at[pl.ds(g * CHUNK, CHUNK)])

        def body(c2, _):
            g = base + c2 * 2
            do_chunk(g, 0)
            do_chunk(g + 1, CHUNK)
            return 0

        lax.fori_loop(0, cpw // 2, body, 0)

    return emb


def kernel(tokens, token_table, position_table):
    b, s = tokens.shape
    _, d_model = token_table.shape
    n_flat = b * s
    emb = _build(n_flat, d_model, s)
    flat_tok = tokens.reshape(n_flat // CHUNK, CHUNK).astype(jnp.int32)
    out = emb(flat_tok, token_table, position_table)
    return out.reshape(b, s, d_model)


# speedup vs baseline: 7.9349x; 1.5228x over previous
import functools

import jax
import jax.numpy as jnp
from jax import lax
from jax.experimental import pallas as pl
from jax.experimental.pallas import tpu as pltpu
from jax.experimental.pallas import tpu_sc as plsc

NC = 2
NS = 16
NW = NC * NS
LANES = 16
GATHER = 100
GPC = 8
CHUNK = GATHER * GPC


def _build(n_flat, d_model, seq_len):
    cpw = n_flat // (NW * CHUNK)
    reps = CHUNK // seq_len
    kd = d_model // LANES
    mesh = plsc.VectorSubcoreMesh(core_axis_name="c", subcore_axis_name="s")

    @functools.partial(
        pl.kernel,
        out_type=jax.ShapeDtypeStruct((n_flat, d_model), jnp.float32),
        mesh=mesh,
        compiler_params=pltpu.CompilerParams(use_tc_tiling_on_sc=False),
        scratch_types=[
            pltpu.VMEM((2, GPC, GATHER), jnp.int32),
            pltpu.VMEM((CHUNK, d_model), jnp.float32),
            pltpu.VMEM((CHUNK, d_model), jnp.float32),
            pltpu.VMEM((seq_len, d_model), jnp.float32),
            pltpu.SemaphoreType.DMA,
            pltpu.SemaphoreType.DMA,
            pltpu.SemaphoreType.DMA,
            pltpu.SemaphoreType.DMA,
        ],
    )
    def emb(tok_hbm, table_hbm, pos_hbm, out_hbm,
            idx_v, rows0, rows1, pos_v, g0, g1, o0, o1):
        rows = (rows0, rows1)
        gsem = (g0, g1)
        wid = lax.axis_index("s") * NC + lax.axis_index("c")
        base = wid * cpw
        pltpu.sync_copy(pos_hbm, pos_v)

        def fire_gathers(g, b):
            pltpu.sync_copy(tok_hbm.at[pl.ds((base + g) * GPC, GPC)],
                            idx_v.at[b])
            for j in range(GPC):
                pltpu.async_copy(
                    table_hbm.at[idx_v.at[b, j]],
                    rows[b].at[pl.ds(j * GATHER, GATHER)],
                    gsem[b],
                )

        def drain(sem, b):
            pltpu.make_async_copy(
                table_hbm.at[pl.ds(0, CHUNK)], rows[b], sem).wait()

        def add_pos(b):
            def body(i, _):
                for r in range(4):
                    row = i * 4 + r
                    for k in range(kd):
                        pv = pos_v[row, pl.ds(k * LANES, LANES)]
                        for t in range(reps):
                            plsc.addupdate(
                                rows[b].at[row + t * seq_len,
                                           pl.ds(k * LANES, LANES)],
                                pv,
                            )
                return 0
            lax.fori_loop(0, seq_len // 4, body, 0)

        fire_gathers(0, 0)

        def pair(p, _):
            g_even = p * 2
            g_odd = g_even + 1

            @pl.when(p >= 1)
            def _():
                drain(o1, 1)
            fire_gathers(g_odd, 1)
            drain(g0, 0)
            add_pos(0)
            pltpu.async_copy(
                rows0, out_hbm.at[pl.ds((base + g_even) * CHUNK, CHUNK)], o0)

            @pl.when(p < cpw // 2 - 1)
            def _():
                drain(o0, 0)
                fire_gathers(g_even + 2, 0)
            drain(g1, 1)
            add_pos(1)
            pltpu.async_copy(
                rows1, out_hbm.at[pl.ds((base + g_odd) * CHUNK, CHUNK)], o1)
            return 0

        lax.fori_loop(0, cpw // 2, pair, 0)
        drain(o0, 0)
        drain(o1, 1)

    return emb


def kernel(tokens, token_table, position_table):
    b, s = tokens.shape
    _, d_model = token_table.shape
    n_flat = b * s
    emb = _build(n_flat, d_model, s)
    flat_tok = tokens.reshape(n_flat // GATHER, GATHER).astype(jnp.int32)
    out = emb(flat_tok, token_table, position_table)
    return out.reshape(b, s, d_model)
